# Initial kernel scaffold; baseline (speedup 1.0000x reference)
import jax
import jax.numpy as jnp
from jax.experimental import pallas as pl


def kernel(x, Wq, bq, Wk, bk, Wv, bv, Wo, bo, ln_g, ln_b, Wg, bg, We, be):
    raise NotImplementedError("write your pallas kernel here")


# speedup vs baseline: 1.5353x; 1.5353x over previous
import jax
import jax.numpy as jnp
from jax.experimental import pallas as pl
from jax.experimental.pallas import tpu as pltpu

D = 1024
H = 16
DH = 64
E = 8
BS = 64
TA = 256
TB = 1024
NEG = -1e30


def _attn_ln_gate_body(x_ref, wq, bq, wk, bk, wv, bv, wo, bo, lg, lb,
                       wg, bg, y_ref, gate_ref, o_scr):
    x = x_ref[...]
    q = jnp.dot(x, wq[...], preferred_element_type=jnp.float32) + bq[...]
    k = jnp.dot(x, wk[...], preferred_element_type=jnp.float32) + bk[...]
    v = jnp.dot(x, wv[...], preferred_element_type=jnp.float32) + bv[...]
    ids = jax.lax.broadcasted_iota(jnp.int32, (TA, TA), 0) // BS
    jds = jax.lax.broadcasted_iota(jnp.int32, (TA, TA), 1) // BS
    mask = ids == jds
    scale = 1.0 / (DH ** 0.5)
    for h in range(H):
        sl = slice(h * DH, (h + 1) * DH)
        qh = q[:, sl]
        kh = k[:, sl]
        vh = v[:, sl]
        s = jax.lax.dot_general(qh, kh, (((1,), (1,)), ((), ())),
                                preferred_element_type=jnp.float32) * scale
        s = jnp.where(mask, s, NEG)
        m = jnp.max(s, axis=-1, keepdims=True)
        p = jnp.exp(s - m)
        p = p / jnp.sum(p, axis=-1, keepdims=True)
        o_scr[:, sl] = jnp.dot(p, vh, preferred_element_type=jnp.float32)
    attn = jnp.dot(o_scr[...], wo[...], preferred_element_type=jnp.float32) + bo[...]
    y = attn + x
    mu = jnp.mean(y, axis=-1, keepdims=True)
    yc = y - mu
    var = jnp.mean(yc * yc, axis=-1, keepdims=True)
    yn = lg[...] * (yc * jax.lax.rsqrt(var + 1e-5)) + lb[...]
    y_ref[...] = yn
    logits = jnp.dot(yn, wg[...], preferred_element_type=jnp.float32) + bg[...]
    mg = jnp.max(logits, axis=-1, keepdims=True)
    eg = jnp.exp(logits - mg)
    gate_ref[...] = eg / jnp.sum(eg, axis=-1, keepdims=True)


def _moe_body(y_ref, gate_ref, we_ref, be_ref, out_ref):
    e = pl.program_id(0)
    w = we_ref[0]
    onehot = (jax.lax.broadcasted_iota(jnp.int32, (1, E), 1) == e).astype(jnp.float32)
    n = y_ref.shape[0]
    for c in range(n // TB):
        rows = slice(c * TB, (c + 1) * TB)
        g = gate_ref[rows, :]
        gcol = jnp.sum(g * onehot, axis=-1, keepdims=True)
        z = y_ref[rows, :] * gcol
        contrib = jnp.dot(z, w, preferred_element_type=jnp.float32) + gcol * be_ref[0]

        @pl.when(e == 0)
        def _():
            out_ref[rows, :] = contrib

        @pl.when(e != 0)
        def _():
            out_ref[rows, :] = out_ref[rows, :] + contrib


def kernel(x, Wq, bq, Wk, bk, Wv, bv, Wo, bo, ln_g, ln_b, Wg, bg, We, be):
    B, S, d = x.shape
    N = B * S
    xf = x.reshape(N, d)
    row = lambda a: a.reshape(1, -1)

    full2 = lambda a, b: pl.BlockSpec((a, b), lambda i: (0, 0))
    y, gate = pl.pallas_call(
        _attn_ln_gate_body,
        grid=(N // TA,),
        in_specs=[
            pl.BlockSpec((TA, d), lambda i: (i, 0)),
            full2(d, d), full2(1, d),
            full2(d, d), full2(1, d),
            full2(d, d), full2(1, d),
            full2(d, d), full2(1, d),
            full2(1, d), full2(1, d),
            full2(d, E), full2(1, E),
        ],
        out_specs=[
            pl.BlockSpec((TA, d), lambda i: (i, 0)),
            pl.BlockSpec((TA, E), lambda i: (i, 0)),
        ],
        out_shape=[
            jax.ShapeDtypeStruct((N, d), jnp.float32),
            jax.ShapeDtypeStruct((N, E), jnp.float32),
        ],
        scratch_shapes=[pltpu.VMEM((TA, d), jnp.float32)],
        compiler_params=pltpu.CompilerParams(
            dimension_semantics=("arbitrary",)),
    )(xf, Wq, row(bq), Wk, row(bk), Wv, row(bv), Wo, row(bo),
      row(ln_g), row(ln_b), Wg, row(bg))

    out = pl.pallas_call(
        _moe_body,
        grid=(E,),
        in_specs=[
            pl.BlockSpec((N, d), lambda e: (0, 0)),
            pl.BlockSpec((N, E), lambda e: (0, 0)),
            pl.BlockSpec((1, d, d), lambda e: (e, 0, 0)),
            pl.BlockSpec((1, 1, d), lambda e: (e, 0, 0)),
        ],
        out_specs=pl.BlockSpec((N, d), lambda e: (0, 0)),
        out_shape=jax.ShapeDtypeStruct((N, d), jnp.float32),
        compiler_params=pltpu.CompilerParams(
            dimension_semantics=("arbitrary",)),
    )(y, gate, We, be.reshape(E, 1, d))
    return out.reshape(B, S, d)
